# baseline (speedup 1.0000x reference)
import functools

import jax
import jax.numpy as jnp
from jax.experimental import pallas as pl

N_LAYERS = 2


def _ggnn_kernel(hin_ref, am_ref, mask_ref, W1_ref, b1_ref, W2T_ref, b2rT_ref,
                 Wi_ref, Wh_ref, bi_ref, bh_ref, Ri1a_ref, Ri1b_ref, ri1_ref,
                 Ri2_ref, ri2_ref, Rj1_ref, rj1_ref, Rj2_ref, rj2_ref,
                 out_ref, *, n, in_size, hid_sz, msg, eh):
    h0 = hin_ref[0]
    amb = am_ref[0]
    msk = mask_ref[0]

    hid = jax.nn.relu(
        jnp.dot(amb, W1_ref[...], preferred_element_type=jnp.float32)
        + b1_ref[...])
    hid2 = hid.reshape(n, n * eh)

    h = jnp.concatenate(
        [h0, jnp.zeros((n, hid_sz - in_size), h0.dtype)], axis=1)

    for _ in range(N_LAYERS):
        P = jnp.dot(h, W2T_ref[...], preferred_element_type=jnp.float32)
        Pm = (P.reshape(n, eh, msg) + 0.0).reshape(n * eh, msg)
        m = jnp.dot(hid2, Pm, preferred_element_type=jnp.float32)
        hsum = jnp.sum(h, axis=0, keepdims=True)
        m = m + jnp.dot(hsum, b2rT_ref[...], preferred_element_type=jnp.float32)
        gi = jnp.dot(m, Wi_ref[...], preferred_element_type=jnp.float32) + bi_ref[...]
        gh = jnp.dot(h, Wh_ref[...], preferred_element_type=jnp.float32) + bh_ref[...]
        r = jax.nn.sigmoid(gi[:, :hid_sz] + gh[:, :hid_sz])
        z = jax.nn.sigmoid(gi[:, hid_sz:2 * hid_sz] + gh[:, hid_sz:2 * hid_sz])
        nn = jnp.tanh(gi[:, 2 * hid_sz:] + r * gh[:, 2 * hid_sz:])
        h = ((1.0 - z) * nn + z * h) * msk

    g1 = jax.nn.relu(
        jnp.dot(h, Ri1a_ref[...], preferred_element_type=jnp.float32)
        + jnp.dot(h0, Ri1b_ref[...], preferred_element_type=jnp.float32)
        + ri1_ref[...])
    gate = jax.nn.sigmoid(
        jnp.dot(g1, Ri2_ref[...], preferred_element_type=jnp.float32)
        + ri2_ref[...])
    val = jnp.dot(
        jax.nn.relu(jnp.dot(h, Rj1_ref[...], preferred_element_type=jnp.float32)
                    + rj1_ref[...]),
        Rj2_ref[...], preferred_element_type=jnp.float32) + rj2_ref[...]
    res = jnp.sum(msk * gate * val, axis=0, keepdims=True)
    mx = jnp.max(res, axis=1, keepdims=True)
    lse = mx + jnp.log(jnp.sum(jnp.exp(res - mx), axis=1, keepdims=True))
    out_ref[0] = res - lse


def kernel(h_in, am, g_size, W1, b1, W2, b2, Wi, Wh, bi, bh,
           Ri1, ri1, Ri2, ri2, Rj1, rj1, Rj2, rj2):
    b, n, in_size = h_in.shape
    e, eh = W1.shape
    hid_sz = Wh.shape[0]
    msg = Wi.shape[0]
    tgt = Ri2.shape[1]

    amf = am.reshape(b, n * n, e)
    W2T = W2.reshape(eh, msg, hid_sz).transpose(2, 0, 1).reshape(hid_sz, eh * msg)
    b2rT = b2.reshape(msg, hid_sz).T
    mask3 = (jnp.arange(n)[None, :] < g_size[:, None]).astype(h_in.dtype)[:, :, None]
    Ri1a, Ri1b = Ri1[:hid_sz], Ri1[hid_sz:]

    row = lambda v: v.reshape(1, -1)
    full = lambda a: pl.BlockSpec(a.shape, lambda i: (0,) * a.ndim)

    weights = [W1, row(b1), W2T, b2rT, Wi, Wh, row(bi), row(bh),
               Ri1a, Ri1b, row(ri1), Ri2, row(ri2), Rj1, row(rj1), Rj2, row(rj2)]

    out = pl.pallas_call(
        functools.partial(_ggnn_kernel, n=n, in_size=in_size, hid_sz=hid_sz,
                          msg=msg, eh=eh),
        grid=(b,),
        in_specs=[
            pl.BlockSpec((1, n, in_size), lambda i: (i, 0, 0)),
            pl.BlockSpec((1, n * n, e), lambda i: (i, 0, 0)),
            pl.BlockSpec((1, n, 1), lambda i: (i, 0, 0)),
        ] + [full(w) for w in weights],
        out_specs=pl.BlockSpec((1, 1, tgt), lambda i: (i, 0, 0)),
        out_shape=jax.ShapeDtypeStruct((b, 1, tgt), h_in.dtype),
    )(h_in, amf, mask3, *weights)
    return out.reshape(b, tgt)


# speedup vs baseline: 1.1682x; 1.1682x over previous
import functools

import jax
import jax.numpy as jnp
from jax.experimental import pallas as pl

N_LAYERS = 2


def _ggnn_kernel(hin_ref, am_ref, mask_ref, W1_ref, b1_ref, W2T_ref, b2rT_ref,
                 Wi_ref, Wh_ref, bi_ref, bh_ref, Ri1a_ref, Ri1b_ref, ri1_ref,
                 Ri2_ref, ri2_ref, Rj1_ref, rj1_ref, Rj2_ref, rj2_ref,
                 out_ref, *, n, in_size, hid_sz, msg, eh):
    h0 = hin_ref[0]
    amb = am_ref[0]
    msk = mask_ref[0]

    hid = jax.nn.relu(
        jax.lax.dot_general(amb, W1_ref[...], (((0,), (0,)), ((), ())),
                            preferred_element_type=jnp.float32)
        + b1_ref[...])
    hid2 = hid.reshape(n, n * eh)

    h = jnp.concatenate(
        [h0, jnp.zeros((n, hid_sz - in_size), h0.dtype)], axis=1)

    for _ in range(N_LAYERS):
        P = jnp.dot(h, W2T_ref[...], preferred_element_type=jnp.float32)
        Pm = (P.reshape(n, eh, msg) + 0.0).reshape(n * eh, msg)
        m = jnp.dot(hid2, Pm, preferred_element_type=jnp.float32)
        hsum = jnp.sum(h, axis=0, keepdims=True)
        m = m + jnp.dot(hsum, b2rT_ref[...], preferred_element_type=jnp.float32)
        gi = jnp.dot(m, Wi_ref[...], preferred_element_type=jnp.float32) + bi_ref[...]
        gh = jnp.dot(h, Wh_ref[...], preferred_element_type=jnp.float32) + bh_ref[...]
        r = jax.nn.sigmoid(gi[:, :hid_sz] + gh[:, :hid_sz])
        z = jax.nn.sigmoid(gi[:, hid_sz:2 * hid_sz] + gh[:, hid_sz:2 * hid_sz])
        nn = jnp.tanh(gi[:, 2 * hid_sz:] + r * gh[:, 2 * hid_sz:])
        h = ((1.0 - z) * nn + z * h) * msk

    g1 = jax.nn.relu(
        jnp.dot(h, Ri1a_ref[...], preferred_element_type=jnp.float32)
        + jnp.dot(h0, Ri1b_ref[...], preferred_element_type=jnp.float32)
        + ri1_ref[...])
    gate = jax.nn.sigmoid(
        jnp.dot(g1, Ri2_ref[...], preferred_element_type=jnp.float32)
        + ri2_ref[...])
    val = jnp.dot(
        jax.nn.relu(jnp.dot(h, Rj1_ref[...], preferred_element_type=jnp.float32)
                    + rj1_ref[...]),
        Rj2_ref[...], preferred_element_type=jnp.float32) + rj2_ref[...]
    res = jnp.sum(msk * gate * val, axis=0, keepdims=True)
    mx = jnp.max(res, axis=1, keepdims=True)
    lse = mx + jnp.log(jnp.sum(jnp.exp(res - mx), axis=1, keepdims=True))
    out_ref[0] = res - lse


def kernel(h_in, am, g_size, W1, b1, W2, b2, Wi, Wh, bi, bh,
           Ri1, ri1, Ri2, ri2, Rj1, rj1, Rj2, rj2):
    b, n, in_size = h_in.shape
    e, eh = W1.shape
    hid_sz = Wh.shape[0]
    msg = Wi.shape[0]
    tgt = Ri2.shape[1]

    amf = am.reshape(b, n * n, e).transpose(0, 2, 1)
    W2T = W2.reshape(eh, msg, hid_sz).transpose(2, 0, 1).reshape(hid_sz, eh * msg)
    b2rT = b2.reshape(msg, hid_sz).T
    mask3 = (jnp.arange(n)[None, :] < g_size[:, None]).astype(h_in.dtype)[:, :, None]
    Ri1a, Ri1b = Ri1[:hid_sz], Ri1[hid_sz:]

    row = lambda v: v.reshape(1, -1)
    full = lambda a: pl.BlockSpec(a.shape, lambda i: (0,) * a.ndim)

    weights = [W1, row(b1), W2T, b2rT, Wi, Wh, row(bi), row(bh),
               Ri1a, Ri1b, row(ri1), Ri2, row(ri2), Rj1, row(rj1), Rj2, row(rj2)]

    out = pl.pallas_call(
        functools.partial(_ggnn_kernel, n=n, in_size=in_size, hid_sz=hid_sz,
                          msg=msg, eh=eh),
        grid=(b,),
        in_specs=[
            pl.BlockSpec((1, n, in_size), lambda i: (i, 0, 0)),
            pl.BlockSpec((1, e, n * n), lambda i: (i, 0, 0)),
            pl.BlockSpec((1, n, 1), lambda i: (i, 0, 0)),
        ] + [full(w) for w in weights],
        out_specs=pl.BlockSpec((1, 1, tgt), lambda i: (i, 0, 0)),
        out_shape=jax.ShapeDtypeStruct((b, 1, tgt), h_in.dtype),
    )(h_in, amf, mask3, *weights)
    return out.reshape(b, tgt)


# speedup vs baseline: 1.4595x; 1.2494x over previous
import functools

import jax
import jax.numpy as jnp
from jax.experimental import pallas as pl

N_LAYERS = 2
G = 4


def _ggnn_kernel(hin_ref, am_ref, mask_ref, W1_ref, b1_ref, W2T_ref, b2rT_ref,
                 Wi_ref, Wh_ref, bi_ref, bh_ref, Ri1a_ref, Ri1b_ref, ri1_ref,
                 Ri2_ref, ri2_ref, Rj1_ref, rj1_ref, Rj2_ref, rj2_ref,
                 out_ref, *, n, in_size, hid_sz, msg, eh):
    h0 = hin_ref[...].reshape(G * n, in_size)
    msk = mask_ref[...].reshape(G * n, 1)

    hid2s = []
    for g in range(G):
        amb = am_ref[g]
        hid = jax.nn.relu(
            jax.lax.dot_general(amb, W1_ref[...], (((0,), (0,)), ((), ())),
                                preferred_element_type=jnp.float32)
            + b1_ref[...])
        hid2s.append(hid.reshape(n, n * eh))

    h = jnp.concatenate(
        [h0, jnp.zeros((G * n, hid_sz - in_size), h0.dtype)], axis=1)

    for _ in range(N_LAYERS):
        P = jnp.dot(h, W2T_ref[...], preferred_element_type=jnp.float32)
        ms = []
        for g in range(G):
            Pg = P[g * n:(g + 1) * n, :]
            Pm = (Pg.reshape(n, eh, msg) + 0.0).reshape(n * eh, msg)
            ms.append(jnp.dot(hid2s[g], Pm, preferred_element_type=jnp.float32))
        m = jnp.concatenate(ms, axis=0)
        hsum = jnp.sum(h.reshape(G, n, hid_sz), axis=1)
        t = jnp.dot(hsum, b2rT_ref[...], preferred_element_type=jnp.float32)
        m = (m.reshape(G, n, msg) + t[:, None, :]).reshape(G * n, msg)
        gi = jnp.dot(m, Wi_ref[...], preferred_element_type=jnp.float32) + bi_ref[...]
        gh = jnp.dot(h, Wh_ref[...], preferred_element_type=jnp.float32) + bh_ref[...]
        r = jax.nn.sigmoid(gi[:, :hid_sz] + gh[:, :hid_sz])
        z = jax.nn.sigmoid(gi[:, hid_sz:2 * hid_sz] + gh[:, hid_sz:2 * hid_sz])
        nn = jnp.tanh(gi[:, 2 * hid_sz:] + r * gh[:, 2 * hid_sz:])
        h = ((1.0 - z) * nn + z * h) * msk

    g1 = jax.nn.relu(
        jnp.dot(h, Ri1a_ref[...], preferred_element_type=jnp.float32)
        + jnp.dot(h0, Ri1b_ref[...], preferred_element_type=jnp.float32)
        + ri1_ref[...])
    gate = jax.nn.sigmoid(
        jnp.dot(g1, Ri2_ref[...], preferred_element_type=jnp.float32)
        + ri2_ref[...])
    val = jnp.dot(
        jax.nn.relu(jnp.dot(h, Rj1_ref[...], preferred_element_type=jnp.float32)
                    + rj1_ref[...]),
        Rj2_ref[...], preferred_element_type=jnp.float32) + rj2_ref[...]
    tgt = val.shape[1]
    res = jnp.sum((msk * gate * val).reshape(G, n, tgt), axis=1)
    mx = jnp.max(res, axis=1, keepdims=True)
    lse = mx + jnp.log(jnp.sum(jnp.exp(res - mx), axis=1, keepdims=True))
    out_ref[...] = (res - lse).reshape(G, 1, tgt)


def kernel(h_in, am, g_size, W1, b1, W2, b2, Wi, Wh, bi, bh,
           Ri1, ri1, Ri2, ri2, Rj1, rj1, Rj2, rj2):
    b, n, in_size = h_in.shape
    e, eh = W1.shape
    hid_sz = Wh.shape[0]
    msg = Wi.shape[0]
    tgt = Ri2.shape[1]

    amf = am.reshape(b, n * n, e).transpose(0, 2, 1)
    W2T = W2.reshape(eh, msg, hid_sz).transpose(2, 0, 1).reshape(hid_sz, eh * msg)
    b2rT = b2.reshape(msg, hid_sz).T
    mask3 = (jnp.arange(n)[None, :] < g_size[:, None]).astype(h_in.dtype)[:, :, None]
    Ri1a, Ri1b = Ri1[:hid_sz], Ri1[hid_sz:]

    row = lambda v: v.reshape(1, -1)
    full = lambda a: pl.BlockSpec(a.shape, lambda i: (0,) * a.ndim)

    weights = [W1, row(b1), W2T, b2rT, Wi, Wh, row(bi), row(bh),
               Ri1a, Ri1b, row(ri1), Ri2, row(ri2), Rj1, row(rj1), Rj2, row(rj2)]

    out = pl.pallas_call(
        functools.partial(_ggnn_kernel, n=n, in_size=in_size, hid_sz=hid_sz,
                          msg=msg, eh=eh),
        grid=(b // G,),
        in_specs=[
            pl.BlockSpec((G, n, in_size), lambda i: (i, 0, 0)),
            pl.BlockSpec((G, e, n * n), lambda i: (i, 0, 0)),
            pl.BlockSpec((G, n, 1), lambda i: (i, 0, 0)),
        ] + [full(w) for w in weights],
        out_specs=pl.BlockSpec((G, 1, tgt), lambda i: (i, 0, 0)),
        out_shape=jax.ShapeDtypeStruct((b, 1, tgt), h_in.dtype),
    )(h_in, amf, mask3, *weights)
    return out.reshape(b, tgt)


# speedup vs baseline: 1.5055x; 1.0315x over previous
import functools

import jax
import jax.numpy as jnp
from jax.experimental import pallas as pl

N_LAYERS = 2
G = 8


def _ggnn_kernel(hin_ref, am_ref, mask_ref, W1_ref, b1_ref, W2T_ref, b2rT_ref,
                 Wi_ref, Wh_ref, bi_ref, bh_ref, Ri1a_ref, Ri1b_ref, ri1_ref,
                 Ri2_ref, ri2_ref, Rj1_ref, rj1_ref, Rj2_ref, rj2_ref,
                 out_ref, *, n, in_size, hid_sz, msg, eh):
    h0 = hin_ref[...].reshape(G * n, in_size)
    msk = mask_ref[...].reshape(G * n, 1)

    hid2s = []
    for g in range(G):
        amb = am_ref[g]
        hid = jax.nn.relu(
            jax.lax.dot_general(amb, W1_ref[...], (((0,), (0,)), ((), ())),
                                preferred_element_type=jnp.float32)
            + b1_ref[...])
        hid2s.append(hid.reshape(n, n * eh))

    h = jnp.concatenate(
        [h0, jnp.zeros((G * n, hid_sz - in_size), h0.dtype)], axis=1)

    for _ in range(N_LAYERS):
        P = jnp.dot(h, W2T_ref[...], preferred_element_type=jnp.float32)
        ms = []
        for g in range(G):
            Pg = P[g * n:(g + 1) * n, :]
            Pm = (Pg.reshape(n, eh, msg) + 0.0).reshape(n * eh, msg)
            ms.append(jnp.dot(hid2s[g], Pm, preferred_element_type=jnp.float32))
        m = jnp.concatenate(ms, axis=0)
        hsum = jnp.sum(h.reshape(G, n, hid_sz), axis=1)
        t = jnp.dot(hsum, b2rT_ref[...], preferred_element_type=jnp.float32)
        m = (m.reshape(G, n, msg) + t[:, None, :]).reshape(G * n, msg)
        gi = jnp.dot(m, Wi_ref[...], preferred_element_type=jnp.float32) + bi_ref[...]
        gh = jnp.dot(h, Wh_ref[...], preferred_element_type=jnp.float32) + bh_ref[...]
        r = jax.nn.sigmoid(gi[:, :hid_sz] + gh[:, :hid_sz])
        z = jax.nn.sigmoid(gi[:, hid_sz:2 * hid_sz] + gh[:, hid_sz:2 * hid_sz])
        nn = jnp.tanh(gi[:, 2 * hid_sz:] + r * gh[:, 2 * hid_sz:])
        h = ((1.0 - z) * nn + z * h) * msk

    g1 = jax.nn.relu(
        jnp.dot(h, Ri1a_ref[...], preferred_element_type=jnp.float32)
        + jnp.dot(h0, Ri1b_ref[...], preferred_element_type=jnp.float32)
        + ri1_ref[...])
    gate = jax.nn.sigmoid(
        jnp.dot(g1, Ri2_ref[...], preferred_element_type=jnp.float32)
        + ri2_ref[...])
    val = jnp.dot(
        jax.nn.relu(jnp.dot(h, Rj1_ref[...], preferred_element_type=jnp.float32)
                    + rj1_ref[...]),
        Rj2_ref[...], preferred_element_type=jnp.float32) + rj2_ref[...]
    tgt = val.shape[1]
    res = jnp.sum((msk * gate * val).reshape(G, n, tgt), axis=1)
    mx = jnp.max(res, axis=1, keepdims=True)
    lse = mx + jnp.log(jnp.sum(jnp.exp(res - mx), axis=1, keepdims=True))
    out_ref[...] = (res - lse).reshape(G, 1, tgt)


def kernel(h_in, am, g_size, W1, b1, W2, b2, Wi, Wh, bi, bh,
           Ri1, ri1, Ri2, ri2, Rj1, rj1, Rj2, rj2):
    b, n, in_size = h_in.shape
    e, eh = W1.shape
    hid_sz = Wh.shape[0]
    msg = Wi.shape[0]
    tgt = Ri2.shape[1]

    amf = am.reshape(b, n * n, e).transpose(0, 2, 1)
    W2T = W2.reshape(eh, msg, hid_sz).transpose(2, 0, 1).reshape(hid_sz, eh * msg)
    b2rT = b2.reshape(msg, hid_sz).T
    mask3 = (jnp.arange(n)[None, :] < g_size[:, None]).astype(h_in.dtype)[:, :, None]
    Ri1a, Ri1b = Ri1[:hid_sz], Ri1[hid_sz:]

    row = lambda v: v.reshape(1, -1)
    full = lambda a: pl.BlockSpec(a.shape, lambda i: (0,) * a.ndim)

    weights = [W1, row(b1), W2T, b2rT, Wi, Wh, row(bi), row(bh),
               Ri1a, Ri1b, row(ri1), Ri2, row(ri2), Rj1, row(rj1), Rj2, row(rj2)]

    out = pl.pallas_call(
        functools.partial(_ggnn_kernel, n=n, in_size=in_size, hid_sz=hid_sz,
                          msg=msg, eh=eh),
        grid=(b // G,),
        in_specs=[
            pl.BlockSpec((G, n, in_size), lambda i: (i, 0, 0)),
            pl.BlockSpec((G, e, n * n), lambda i: (i, 0, 0)),
            pl.BlockSpec((G, n, 1), lambda i: (i, 0, 0)),
        ] + [full(w) for w in weights],
        out_specs=pl.BlockSpec((G, 1, tgt), lambda i: (i, 0, 0)),
        out_shape=jax.ShapeDtypeStruct((b, 1, tgt), h_in.dtype),
    )(h_in, amf, mask3, *weights)
    return out.reshape(b, tgt)


# speedup vs baseline: 1.8322x; 1.2170x over previous
import functools

import jax
import jax.numpy as jnp
from jax.experimental import pallas as pl

N_LAYERS = 2
G = 8


def _ggnn_kernel(hin_ref, am_ref, mask_ref, W1_ref, b1_ref, W2T_ref, b2rT_ref,
                 Wi_ref, Wh_ref, bi_ref, bh_ref, Ri1a_ref, Ri1b_ref, ri1_ref,
                 Ri2_ref, ri2_ref, Rj1_ref, rj1_ref, Rj2_ref, rj2_ref,
                 out_ref, *, n, in_size, hid_sz, msg, eh):
    h0 = hin_ref[...].reshape(G * n, in_size)
    msk = mask_ref[...].reshape(G * n, 1)

    hid2s = []
    for g in range(G):
        amb = am_ref[g]
        hid = jax.nn.relu(
            jax.lax.dot_general(amb, W1_ref[...], (((0,), (0,)), ((), ())),
                                preferred_element_type=jnp.float32)
            + b1_ref[...])
        hid2s.append(hid.astype(jnp.bfloat16).reshape(n, n * eh))

    h = jnp.concatenate(
        [h0, jnp.zeros((G * n, hid_sz - in_size), h0.dtype)], axis=1)

    for _ in range(N_LAYERS):
        P = jnp.dot(h.astype(jnp.bfloat16), W2T_ref[...],
                    preferred_element_type=jnp.float32).astype(jnp.bfloat16)
        ms = []
        for g in range(G):
            Pg = P[g * n:(g + 1) * n, :]
            Pm = (Pg.reshape(n, eh, msg) + jnp.bfloat16(0.0)).reshape(n * eh, msg)
            ms.append(jnp.dot(hid2s[g], Pm, preferred_element_type=jnp.float32))
        m = jnp.concatenate(ms, axis=0)
        hsum = jnp.sum(h.reshape(G, n, hid_sz), axis=1)
        t = jnp.dot(hsum, b2rT_ref[...], preferred_element_type=jnp.float32)
        m = (m.reshape(G, n, msg) + t[:, None, :]).reshape(G * n, msg)
        gi = jnp.dot(m, Wi_ref[...], preferred_element_type=jnp.float32) + bi_ref[...]
        gh = jnp.dot(h, Wh_ref[...], preferred_element_type=jnp.float32) + bh_ref[...]
        r = jax.nn.sigmoid(gi[:, :hid_sz] + gh[:, :hid_sz])
        z = jax.nn.sigmoid(gi[:, hid_sz:2 * hid_sz] + gh[:, hid_sz:2 * hid_sz])
        nn = jnp.tanh(gi[:, 2 * hid_sz:] + r * gh[:, 2 * hid_sz:])
        h = ((1.0 - z) * nn + z * h) * msk

    g1 = jax.nn.relu(
        jnp.dot(h, Ri1a_ref[...], preferred_element_type=jnp.float32)
        + jnp.dot(h0, Ri1b_ref[...], preferred_element_type=jnp.float32)
        + ri1_ref[...])
    gate = jax.nn.sigmoid(
        jnp.dot(g1, Ri2_ref[...], preferred_element_type=jnp.float32)
        + ri2_ref[...])
    val = jnp.dot(
        jax.nn.relu(jnp.dot(h, Rj1_ref[...], preferred_element_type=jnp.float32)
                    + rj1_ref[...]),
        Rj2_ref[...], preferred_element_type=jnp.float32) + rj2_ref[...]
    tgt = val.shape[1]
    res = jnp.sum((msk * gate * val).reshape(G, n, tgt), axis=1)
    mx = jnp.max(res, axis=1, keepdims=True)
    lse = mx + jnp.log(jnp.sum(jnp.exp(res - mx), axis=1, keepdims=True))
    out_ref[...] = (res - lse).reshape(G, 1, tgt)


def kernel(h_in, am, g_size, W1, b1, W2, b2, Wi, Wh, bi, bh,
           Ri1, ri1, Ri2, ri2, Rj1, rj1, Rj2, rj2):
    b, n, in_size = h_in.shape
    e, eh = W1.shape
    hid_sz = Wh.shape[0]
    msg = Wi.shape[0]
    tgt = Ri2.shape[1]

    amf = am.reshape(b, n * n, e).transpose(0, 2, 1).astype(jnp.bfloat16)
    W2T = W2.reshape(eh, msg, hid_sz).transpose(2, 0, 1).reshape(hid_sz, eh * msg)
    W2T = W2T.astype(jnp.bfloat16)
    b2rT = b2.reshape(msg, hid_sz).T
    mask3 = (jnp.arange(n)[None, :] < g_size[:, None]).astype(h_in.dtype)[:, :, None]
    Ri1a, Ri1b = Ri1[:hid_sz], Ri1[hid_sz:]

    row = lambda v: v.reshape(1, -1)
    full = lambda a: pl.BlockSpec(a.shape, lambda i: (0,) * a.ndim)

    weights = [W1.astype(jnp.bfloat16), row(b1), W2T, b2rT, Wi, Wh, row(bi), row(bh),
               Ri1a, Ri1b, row(ri1), Ri2, row(ri2), Rj1, row(rj1), Rj2, row(rj2)]

    out = pl.pallas_call(
        functools.partial(_ggnn_kernel, n=n, in_size=in_size, hid_sz=hid_sz,
                          msg=msg, eh=eh),
        grid=(b // G,),
        in_specs=[
            pl.BlockSpec((G, n, in_size), lambda i: (i, 0, 0)),
            pl.BlockSpec((G, e, n * n), lambda i: (i, 0, 0)),
            pl.BlockSpec((G, n, 1), lambda i: (i, 0, 0)),
        ] + [full(w) for w in weights],
        out_specs=pl.BlockSpec((G, 1, tgt), lambda i: (i, 0, 0)),
        out_shape=jax.ShapeDtypeStruct((b, 1, tgt), h_in.dtype),
    )(h_in, amf, mask3, *weights)
    return out.reshape(b, tgt)


# speedup vs baseline: 1.9141x; 1.0447x over previous
import functools

import jax
import jax.numpy as jnp
from jax.experimental import pallas as pl

N_LAYERS = 2
G = 8


def _ggnn_kernel(hin_ref, am_ref, mask_ref, W1T3_ref, b1_ref, W2T_ref,
                 b2rT_ref, Wi_ref, Wh_ref, bi_ref, bh_ref, Ri1a_ref, Ri1b_ref,
                 ri1_ref, Ri2_ref, ri2_ref, Rj1_ref, rj1_ref, Rj2_ref,
                 rj2_ref, out_ref, *, n, in_size, hid_sz, msg, eh):
    h0 = hin_ref[...].reshape(G * n, in_size)
    msk = mask_ref[...].reshape(G * n, 1)

    b13 = b1_ref[...][None, :, :]
    hid2s = []
    for g in range(G):
        amv = am_ref[g]
        hidY = jax.lax.dot_general(
            W1T3_ref[...], amv, (((2,), (1,)), ((0,), (0,))),
            preferred_element_type=jnp.float32)
        hidY = jax.nn.relu(hidY + b13)
        hid2s.append(hidY.astype(jnp.bfloat16).reshape(n, n * eh))

    h = jnp.concatenate(
        [h0, jnp.zeros((G * n, hid_sz - in_size), h0.dtype)], axis=1)

    for _ in range(N_LAYERS):
        P = jnp.dot(h.astype(jnp.bfloat16), W2T_ref[...],
                    preferred_element_type=jnp.float32).astype(jnp.bfloat16)
        ms = []
        for g in range(G):
            Pm = jnp.concatenate(
                [P[g * n:(g + 1) * n, k * msg:(k + 1) * msg]
                 for k in range(eh)], axis=0)
            ms.append(jnp.dot(hid2s[g], Pm, preferred_element_type=jnp.float32))
        m = jnp.concatenate(ms, axis=0)
        hsum = jnp.sum(h.reshape(G, n, hid_sz), axis=1)
        t = jnp.dot(hsum, b2rT_ref[...], preferred_element_type=jnp.float32)
        m = (m.reshape(G, n, msg) + t[:, None, :]).reshape(G * n, msg)
        gi = jnp.dot(m, Wi_ref[...], preferred_element_type=jnp.float32) + bi_ref[...]
        gh = jnp.dot(h, Wh_ref[...], preferred_element_type=jnp.float32) + bh_ref[...]
        r = jax.nn.sigmoid(gi[:, :hid_sz] + gh[:, :hid_sz])
        z = jax.nn.sigmoid(gi[:, hid_sz:2 * hid_sz] + gh[:, hid_sz:2 * hid_sz])
        nn = jnp.tanh(gi[:, 2 * hid_sz:] + r * gh[:, 2 * hid_sz:])
        h = ((1.0 - z) * nn + z * h) * msk

    g1 = jax.nn.relu(
        jnp.dot(h, Ri1a_ref[...], preferred_element_type=jnp.float32)
        + jnp.dot(h0, Ri1b_ref[...], preferred_element_type=jnp.float32)
        + ri1_ref[...])
    gate = jax.nn.sigmoid(
        jnp.dot(g1, Ri2_ref[...], preferred_element_type=jnp.float32)
        + ri2_ref[...])
    val = jnp.dot(
        jax.nn.relu(jnp.dot(h, Rj1_ref[...], preferred_element_type=jnp.float32)
                    + rj1_ref[...]),
        Rj2_ref[...], preferred_element_type=jnp.float32) + rj2_ref[...]
    tgt = val.shape[1]
    res = jnp.sum((msk * gate * val).reshape(G, n, tgt), axis=1)
    mx = jnp.max(res, axis=1, keepdims=True)
    lse = mx + jnp.log(jnp.sum(jnp.exp(res - mx), axis=1, keepdims=True))
    out_ref[...] = (res - lse).reshape(G, 1, tgt)


def kernel(h_in, am, g_size, W1, b1, W2, b2, Wi, Wh, bi, bh,
           Ri1, ri1, Ri2, ri2, Rj1, rj1, Rj2, rj2):
    b, n, in_size = h_in.shape
    e, eh = W1.shape
    hid_sz = Wh.shape[0]
    msg = Wi.shape[0]
    tgt = Ri2.shape[1]

    amp = am.transpose(0, 1, 3, 2).astype(jnp.bfloat16)
    W1T3 = jnp.broadcast_to(W1.T.astype(jnp.bfloat16)[None], (n, eh, e))
    b1c = b1.reshape(eh, 1)
    W2T = W2.reshape(eh, msg, hid_sz).transpose(2, 0, 1).reshape(hid_sz, eh * msg)
    W2T = W2T.astype(jnp.bfloat16)
    b2rT = b2.reshape(msg, hid_sz).T
    mask3 = (jnp.arange(n)[None, :] < g_size[:, None]).astype(h_in.dtype)[:, :, None]
    Ri1a, Ri1b = Ri1[:hid_sz], Ri1[hid_sz:]

    row = lambda v: v.reshape(1, -1)
    full = lambda a: pl.BlockSpec(a.shape, lambda i: (0,) * a.ndim)

    weights = [W1T3, b1c, W2T, b2rT, Wi, Wh, row(bi), row(bh),
               Ri1a, Ri1b, row(ri1), Ri2, row(ri2), Rj1, row(rj1), Rj2, row(rj2)]

    out = pl.pallas_call(
        functools.partial(_ggnn_kernel, n=n, in_size=in_size, hid_sz=hid_sz,
                          msg=msg, eh=eh),
        grid=(b // G,),
        in_specs=[
            pl.BlockSpec((G, n, in_size), lambda i: (i, 0, 0)),
            pl.BlockSpec((G, n, e, n), lambda i: (i, 0, 0, 0)),
            pl.BlockSpec((G, n, 1), lambda i: (i, 0, 0)),
        ] + [full(w) for w in weights],
        out_specs=pl.BlockSpec((G, 1, tgt), lambda i: (i, 0, 0)),
        out_shape=jax.ShapeDtypeStruct((b, 1, tgt), h_in.dtype),
    )(h_in, amp, mask3, *weights)
    return out.reshape(b, tgt)
